# baseline (speedup 1.0000x reference)
import jax
import jax.numpy as jnp
from jax import lax
from jax.experimental import pallas as pl
from jax.experimental.pallas import tpu as pltpu

_K_MAX = 16
_LAMBDA = 1000.0
_MAX_ITER = 50
_CHUNK = 1000


def _dot(a, b, dims, precision=lax.Precision.HIGHEST):
    return lax.dot_general(
        a, b, dimension_numbers=(dims, ((), ())),
        preferred_element_type=jnp.float32,
        precision=precision,
    )


def _dpmeans_kernel(x_hbm_ref, mu_out_ref, x_ref, copy_sems):
    n, d = x_hbm_ref.shape
    n_chunks = n // _CHUNK
    iota32 = lax.broadcasted_iota(jnp.int32, (1, 2 * _K_MAX), 1)
    iota16 = lax.broadcasted_iota(jnp.int32, (1, _K_MAX), 1)
    iota16c = lax.broadcasted_iota(jnp.int32, (_K_MAX, 1), 0)
    row0 = (iota16c == 0).astype(jnp.float32)
    ones_c = jnp.ones((_CHUNK, 1), jnp.float32)

    def chunk_copy(c):
        return pltpu.make_async_copy(
            x_hbm_ref.at[pl.ds(c * _CHUNK, _CHUNK), :],
            x_ref.at[pl.ds(c * _CHUNK, _CHUNK), :],
            copy_sems.at[c])

    for c in range(n_chunks):
        chunk_copy(c).start()
    ones_d = jnp.ones((1, d), jnp.float32)
    total = jnp.zeros((1, d), jnp.float32)
    macc = jnp.zeros((1, _CHUNK), jnp.float32)
    for c in range(n_chunks):
        chunk_copy(c).wait()
        xc = x_ref[pl.ds(c * _CHUNK, _CHUNK), :]
        total = total + jnp.sum(xc, axis=0, keepdims=True)
        sq = _dot(ones_d, xc * xc, ((1,), (1,)), lax.Precision.DEFAULT)
        macc = jnp.maximum(macc, sq)
    maxsq = jnp.max(macc)
    mu0 = row0 * (total / jnp.float32(n))

    msq = jnp.sum((total / jnp.float32(n)) ** 2)
    certified = (jnp.sqrt(maxsq) + jnp.sqrt(msq)) ** 2 < 0.9 * _LAMBDA

    mu0_sq = _dot(jnp.ones((1, d), jnp.float32), mu0 * mu0, ((1,), (1,)))

    def probe_body(c, far_cnt):
        xc = x_ref[pl.ds(c * _CHUNK, _CHUNK), :]
        x_sq = jnp.sum(xc * xc, axis=1, keepdims=True)
        prod = _dot(xc, mu0, ((1,), (1,)), lax.Precision.DEFAULT)
        dmin = x_sq - 2.0 * prod[:, 0:1] + mu0_sq[:, 0:1]
        return far_cnt + jnp.sum((dmin > _LAMBDA).astype(jnp.float32))

    n_probe = jnp.where(certified, 0, n_chunks)
    far_cnt0 = lax.fori_loop(0, n_probe, probe_body, jnp.float32(0.0))

    def sweep(mu, k):
        mu_sq = _dot(jnp.ones((1, d), jnp.float32), mu * mu, ((1,), (1,)))

        def chunk_body(c, acc):
            sums, counts = acc
            xc = x_ref[pl.ds(c * _CHUNK, _CHUNK), :]
            x_sq = jnp.sum(xc * xc, axis=1, keepdims=True)
            prod = _dot(xc, mu, ((1,), (1,)))
            dist = x_sq - 2.0 * prod + mu_sq
            dist_m = jnp.where(iota16 < k, dist, jnp.inf)
            dmin = jnp.min(dist_m, axis=1, keepdims=True)
            eq = dist_m == dmin
            z = jnp.min(jnp.where(eq, iota16, _K_MAX), axis=1,
                        keepdims=True)
            far = dmin > _LAMBDA
            col = jnp.where(far, iota32 - _K_MAX, iota32)
            onehot = (col == z).astype(jnp.float32)
            sums = sums + _dot(onehot, xc, ((0,), (0,)))
            counts = counts + _dot(onehot, ones_c, ((0,), (0,)))
            return sums, counts

        init = (jnp.zeros((2 * _K_MAX, d), jnp.float32),
                jnp.zeros((2 * _K_MAX, 1), jnp.float32))
        sums32, counts32 = lax.fori_loop(0, n_chunks, chunk_body, init)

        far_cnt = jnp.sum(counts32[_K_MAX:])
        grow = jnp.logical_and(far_cnt > 0.0, k < _K_MAX)
        far_sum = jnp.sum(sums32[_K_MAX:], axis=0, keepdims=True)
        ek = (iota16c == k).astype(jnp.float32)
        sums = jnp.where(grow,
                         sums32[:_K_MAX] + ek * far_sum,
                         sums32[:_K_MAX] + sums32[_K_MAX:])
        counts = jnp.where(grow,
                           counts32[:_K_MAX] + ek * far_cnt,
                           counts32[:_K_MAX] + counts32[_K_MAX:])
        mu_new = jnp.where(counts > 0.0,
                           sums / jnp.maximum(counts, 1.0), mu)
        return mu_new, k + grow.astype(jnp.int32)

    def body(state):
        mu, k, _done, it = state
        mu_new, k_new = sweep(mu, k)
        delta = jnp.sum(jnp.abs(mu_new - mu))
        done = jnp.logical_and(delta == 0.0, k_new == k)
        return mu_new, k_new, done, it + 1

    def cond(state):
        _mu, _k, done, it = state
        return jnp.logical_and(it < _MAX_ITER, jnp.logical_not(done))

    state0 = (mu0, jnp.int32(1), far_cnt0 == 0.0, jnp.int32(0))
    mu_fin, _, _, _ = lax.while_loop(cond, body, state0)
    mu_out_ref[...] = mu_fin


def kernel(x):
    X = x[0]
    n, d = X.shape
    mu = pl.pallas_call(
        _dpmeans_kernel,
        out_shape=jax.ShapeDtypeStruct((_K_MAX, d), jnp.float32),
        in_specs=[pl.BlockSpec(memory_space=pltpu.MemorySpace.HBM)],
        out_specs=pl.BlockSpec((_K_MAX, d), lambda: (0, 0)),
        scratch_shapes=[
            pltpu.VMEM((n, d), jnp.float32),
            pltpu.SemaphoreType.DMA((n // _CHUNK,)),
        ],
        compiler_params=pltpu.CompilerParams(
            vmem_limit_bytes=60 * 1024 * 1024,
        ),
    )(X)
    return mu[None, :, :]


# speedup vs baseline: 1.4028x; 1.4028x over previous
import jax
import jax.numpy as jnp
from jax import lax
from jax.experimental import pallas as pl
from jax.experimental.pallas import tpu as pltpu

_K_MAX = 16
_LAMBDA = 1000.0
_MAX_ITER = 50
_CHUNK = 2000


def _dot(a, b, dims, precision=lax.Precision.HIGHEST):
    return lax.dot_general(
        a, b, dimension_numbers=(dims, ((), ())),
        preferred_element_type=jnp.float32,
        precision=precision,
    )


def _dpmeans_kernel(x_hbm_ref, mu_out_ref, x_ref, copy_sems):
    n, d = x_hbm_ref.shape
    n_chunks = n // _CHUNK
    iota32 = lax.broadcasted_iota(jnp.int32, (1, 2 * _K_MAX), 1)
    iota16 = lax.broadcasted_iota(jnp.int32, (1, _K_MAX), 1)
    iota16c = lax.broadcasted_iota(jnp.int32, (_K_MAX, 1), 0)
    row0 = (iota16c == 0).astype(jnp.float32)
    ones_c = jnp.ones((_CHUNK, 1), jnp.float32)

    def chunk_copy(c):
        return pltpu.make_async_copy(
            x_hbm_ref.at[pl.ds(c * _CHUNK, _CHUNK), :],
            x_ref.at[pl.ds(c * _CHUNK, _CHUNK), :],
            copy_sems.at[c])

    for c in range(n_chunks):
        chunk_copy(c).start()
    ones_d = jnp.ones((1, d), jnp.float32)
    total = jnp.zeros((1, d), jnp.float32)
    macc = jnp.zeros((1, _CHUNK), jnp.float32)
    for c in range(n_chunks):
        chunk_copy(c).wait()
        xc = x_ref[pl.ds(c * _CHUNK, _CHUNK), :]
        total = total + jnp.sum(xc, axis=0, keepdims=True)
        sq = _dot(ones_d, xc * xc, ((1,), (1,)), lax.Precision.DEFAULT)
        macc = jnp.maximum(macc, sq)
    maxsq = jnp.max(macc)
    mu0 = row0 * (total / jnp.float32(n))

    msq = jnp.sum((total / jnp.float32(n)) ** 2)
    certified = (jnp.sqrt(maxsq) + jnp.sqrt(msq)) ** 2 < 0.9 * _LAMBDA

    mu0_sq = _dot(jnp.ones((1, d), jnp.float32), mu0 * mu0, ((1,), (1,)))

    def probe_body(c, far_cnt):
        xc = x_ref[pl.ds(c * _CHUNK, _CHUNK), :]
        x_sq = jnp.sum(xc * xc, axis=1, keepdims=True)
        prod = _dot(xc, mu0, ((1,), (1,)), lax.Precision.DEFAULT)
        dmin = x_sq - 2.0 * prod[:, 0:1] + mu0_sq[:, 0:1]
        return far_cnt + jnp.sum((dmin > _LAMBDA).astype(jnp.float32))

    n_probe = jnp.where(certified, 0, n_chunks)
    far_cnt0 = lax.fori_loop(0, n_probe, probe_body, jnp.float32(0.0))

    def sweep(mu, k):
        mu_sq = _dot(jnp.ones((1, d), jnp.float32), mu * mu, ((1,), (1,)))

        def chunk_body(c, acc):
            sums, counts = acc
            xc = x_ref[pl.ds(c * _CHUNK, _CHUNK), :]
            x_sq = jnp.sum(xc * xc, axis=1, keepdims=True)
            prod = _dot(xc, mu, ((1,), (1,)))
            dist = x_sq - 2.0 * prod + mu_sq
            dist_m = jnp.where(iota16 < k, dist, jnp.inf)
            dmin = jnp.min(dist_m, axis=1, keepdims=True)
            eq = dist_m == dmin
            z = jnp.min(jnp.where(eq, iota16, _K_MAX), axis=1,
                        keepdims=True)
            far = dmin > _LAMBDA
            col = jnp.where(far, iota32 - _K_MAX, iota32)
            onehot = (col == z).astype(jnp.float32)
            sums = sums + _dot(onehot, xc, ((0,), (0,)))
            counts = counts + _dot(onehot, ones_c, ((0,), (0,)))
            return sums, counts

        init = (jnp.zeros((2 * _K_MAX, d), jnp.float32),
                jnp.zeros((2 * _K_MAX, 1), jnp.float32))
        sums32, counts32 = lax.fori_loop(0, n_chunks, chunk_body, init)

        far_cnt = jnp.sum(counts32[_K_MAX:])
        grow = jnp.logical_and(far_cnt > 0.0, k < _K_MAX)
        far_sum = jnp.sum(sums32[_K_MAX:], axis=0, keepdims=True)
        ek = (iota16c == k).astype(jnp.float32)
        sums = jnp.where(grow,
                         sums32[:_K_MAX] + ek * far_sum,
                         sums32[:_K_MAX] + sums32[_K_MAX:])
        counts = jnp.where(grow,
                           counts32[:_K_MAX] + ek * far_cnt,
                           counts32[:_K_MAX] + counts32[_K_MAX:])
        mu_new = jnp.where(counts > 0.0,
                           sums / jnp.maximum(counts, 1.0), mu)
        return mu_new, k + grow.astype(jnp.int32)

    def body(state):
        mu, k, _done, it = state
        mu_new, k_new = sweep(mu, k)
        delta = jnp.sum(jnp.abs(mu_new - mu))
        done = jnp.logical_and(delta == 0.0, k_new == k)
        return mu_new, k_new, done, it + 1

    def cond(state):
        _mu, _k, done, it = state
        return jnp.logical_and(it < _MAX_ITER, jnp.logical_not(done))

    state0 = (mu0, jnp.int32(1), far_cnt0 == 0.0, jnp.int32(0))
    mu_fin, _, _, _ = lax.while_loop(cond, body, state0)
    mu_out_ref[...] = mu_fin


def kernel(x):
    X = x[0]
    n, d = X.shape
    mu = pl.pallas_call(
        _dpmeans_kernel,
        out_shape=jax.ShapeDtypeStruct((_K_MAX, d), jnp.float32),
        in_specs=[pl.BlockSpec(memory_space=pltpu.MemorySpace.HBM)],
        out_specs=pl.BlockSpec((_K_MAX, d), lambda: (0, 0)),
        scratch_shapes=[
            pltpu.VMEM((n, d), jnp.float32),
            pltpu.SemaphoreType.DMA((n // _CHUNK,)),
        ],
        compiler_params=pltpu.CompilerParams(
            vmem_limit_bytes=60 * 1024 * 1024,
        ),
    )(X)
    return mu[None, :, :]
